# Initial kernel scaffold; baseline (speedup 1.0000x reference)
import jax
import jax.numpy as jnp
from jax.experimental import pallas as pl


def kernel(tweet, edge_index, edge_type, W_tweet, b_tweet, W_in, b_in, W_rel, W_root, b_rgcn, W_out, b_out, W_fin, b_fin):
    raise NotImplementedError("write your pallas kernel here")


# speedup vs baseline: 5.1288x; 5.1288x over previous
import jax
import jax.numpy as jnp
from jax import lax
from jax.experimental import pallas as pl
from jax.experimental.pallas import tpu as pltpu
from jax.experimental.pallas import tpu_sc as plsc

N = 10000
E = 320000
D = 128
T = 768
R = 2
H = D // 2
NC = 2
NS = 16
L = 16
NP = 10240
K = 80
SP_A = 256
SP_C = 128
HS = SP_A // 2
RPT = (R * NP) // NS
TRASH = N
BN = 1000
GRID = N // BN


def _leaky(x):
    return jnp.where(x > 0, x, 0.01 * x)


def _sc_agg_body(xcat, srcb, dstb, z64, agg_o, src_v, dst_v, rows_v, acc_sp):
    cid = lax.axis_index("c")
    sid = lax.axis_index("s")
    row0 = sid * RPT

    pltpu.sync_copy(z64, acc_sp.at[pl.ds(row0, RPT)])
    plsc.subcore_barrier()

    for h in range(2):
        pltpu.sync_copy(srcb.at[cid, sid, pl.ds(h * HS, HS)], src_v)
        pltpu.sync_copy(dstb.at[sid, pl.ds(h * HS, HS)], dst_v)

        def step(j, c):
            pltpu.sync_copy(xcat.at[src_v.at[j]], rows_v)
            pltpu.sync_copy(rows_v, acc_sp.at[dst_v.at[j]], add=True)
            return c
        lax.fori_loop(0, HS, step, 0)

    plsc.subcore_barrier()

    pltpu.sync_copy(acc_sp.at[pl.ds(row0, RPT)],
                    agg_o.at[cid, pl.ds(row0, RPT)])


_sc_agg = pl.kernel(
    _sc_agg_body,
    out_type=jax.ShapeDtypeStruct((NC, R * NP, H), jnp.float32),
    mesh=plsc.VectorSubcoreMesh(core_axis_name="c", subcore_axis_name="s"),
    scratch_types=(
        pltpu.VMEM((HS, K), jnp.int32),
        pltpu.VMEM((HS, K), jnp.int32),
        pltpu.VMEM((K, H), jnp.float32),
        pltpu.VMEM_SHARED((R * NP, H), jnp.float32),
    ),
    compiler_params=pltpu.CompilerParams(use_tc_tiling_on_sc=False),
)


def _sc_cnt_body(dstc, ones_h, z16, cnt_o, dst_v, ones_v, cnt_sp):
    cid = lax.axis_index("c")
    sid = lax.axis_index("s")
    row0 = sid * RPT

    pltpu.sync_copy(z16, cnt_sp.at[pl.ds(row0, RPT)])
    pltpu.sync_copy(ones_h, ones_v)
    pltpu.sync_copy(dstc.at[cid, sid], dst_v)
    plsc.subcore_barrier()

    def step(j, c):
        pltpu.sync_copy(ones_v, cnt_sp.at[dst_v.at[j]], add=True)
        return c
    lax.fori_loop(0, SP_C, step, 0)

    plsc.subcore_barrier()
    pltpu.sync_copy(cnt_sp.at[pl.ds(row0, RPT)],
                    cnt_o.at[cid, pl.ds(row0, RPT)])


_sc_cnt = pl.kernel(
    _sc_cnt_body,
    out_type=jax.ShapeDtypeStruct((NC, R * NP, L), jnp.float32),
    mesh=plsc.VectorSubcoreMesh(core_axis_name="c", subcore_axis_name="s"),
    scratch_types=(
        pltpu.VMEM((SP_C, K), jnp.int32),
        pltpu.VMEM((K, L), jnp.float32),
        pltpu.VMEM_SHARED((R * NP, L), jnp.float32),
    ),
    compiler_params=pltpu.CompilerParams(use_tc_tiling_on_sc=False),
)


def _tc_input(tweet, Wt, bt, Wi, bi):
    def body(tw, wt, bt_, wi, bi_, out):
        x = jnp.dot(tw[...], wt[...], preferred_element_type=jnp.float32)
        x = _leaky(x + bt_[...])
        x = jnp.dot(x, wi[...], preferred_element_type=jnp.float32)
        x = _leaky(x + bi_[...])
        out[0] = x[:, :H]
        out[1] = x[:, H:]

    return pl.pallas_call(
        body,
        grid=(GRID,),
        in_specs=[
            pl.BlockSpec((BN, T), lambda i: (i, 0)),
            pl.BlockSpec((T, D), lambda i: (0, 0)),
            pl.BlockSpec((1, D), lambda i: (0, 0)),
            pl.BlockSpec((D, D), lambda i: (0, 0)),
            pl.BlockSpec((1, D), lambda i: (0, 0)),
        ],
        out_specs=pl.BlockSpec((2, BN, H), lambda i: (0, i, 0)),
        out_shape=jax.ShapeDtypeStruct((2, N, H), jnp.float32),
    )(tweet, Wt, bt, Wi, bi)


def _tc_combine(xs, agg, cnt, Wroot2, Wrel2, b):
    def body(x, a, c, wr, wl, bb, out):
        acc = (jnp.dot(x[0], wr[0], preferred_element_type=jnp.float32)
               + jnp.dot(x[1], wr[1], preferred_element_type=jnp.float32)
               + bb[...])
        for r in range(R):
            s = (jnp.dot(a[0, r], wl[0, r], preferred_element_type=jnp.float32)
                 + jnp.dot(a[1, r], wl[1, r], preferred_element_type=jnp.float32))
            den = c[0, r, :, 0:1] + c[1, r, :, 0:1]
            acc = acc + s / jnp.maximum(den, 1.0)
        out[0] = acc[:, :H]
        out[1] = acc[:, H:]

    return pl.pallas_call(
        body,
        grid=(GRID,),
        in_specs=[
            pl.BlockSpec((2, BN, H), lambda i: (0, i, 0)),
            pl.BlockSpec((2, R, BN, H), lambda i: (0, 0, i, 0)),
            pl.BlockSpec((2, R, BN, L), lambda i: (0, 0, i, 0)),
            pl.BlockSpec((2, H, D), lambda i: (0, 0, 0)),
            pl.BlockSpec((2, R, H, D), lambda i: (0, 0, 0, 0)),
            pl.BlockSpec((1, D), lambda i: (0, 0)),
        ],
        out_specs=pl.BlockSpec((2, BN, H), lambda i: (0, i, 0)),
        out_shape=jax.ShapeDtypeStruct((2, N, H), jnp.float32),
    )(xs, agg, cnt, Wroot2, Wrel2, b)


def _tc_final(xs, Wout2, bo, Wfin, bf):
    def body(x, wo, bo_, wf, bf_, out):
        h = (jnp.dot(x[0], wo[0], preferred_element_type=jnp.float32)
             + jnp.dot(x[1], wo[1], preferred_element_type=jnp.float32)
             + bo_[...])
        h = _leaky(h)
        out[...] = jnp.dot(h, wf[...], preferred_element_type=jnp.float32) + bf_[...]

    return pl.pallas_call(
        body,
        grid=(GRID,),
        in_specs=[
            pl.BlockSpec((2, BN, H), lambda i: (0, i, 0)),
            pl.BlockSpec((2, H, D), lambda i: (0, 0, 0)),
            pl.BlockSpec((1, D), lambda i: (0, 0)),
            pl.BlockSpec((D, 2), lambda i: (0, 0)),
            pl.BlockSpec((1, 2), lambda i: (0, 0)),
        ],
        out_specs=pl.BlockSpec((BN, 2), lambda i: (i, 0)),
        out_shape=jax.ShapeDtypeStruct((N, 2), jnp.float32),
    )(xs, Wout2, bo, Wfin, bf)


def kernel(tweet, edge_index, edge_type, W_tweet, b_tweet, W_in, b_in,
           W_rel, W_root, b_rgcn, W_out, b_out, W_fin, b_fin):
    src = edge_index[0].astype(jnp.int32)
    dstp = edge_index[1].astype(jnp.int32) + edge_type.astype(jnp.int32) * NP

    srcp = src.reshape(NS, SP_A - 6, K)
    srcp = jnp.pad(srcp, ((0, 0), (0, 6), (0, 0)))
    srcb = srcp[None, :, :, :] + (jnp.arange(NC, dtype=jnp.int32) * N
                                  ).reshape(NC, 1, 1, 1)
    dstb = dstp.reshape(NS, SP_A - 6, K)
    dstb = jnp.pad(dstb, ((0, 0), (0, 6), (0, 0)), constant_values=TRASH)

    dstc = dstp.reshape(NC, NS, SP_C - 3, K)
    dstc = jnp.pad(dstc, ((0, 0), (0, 0), (0, 3), (0, 0)),
                   constant_values=TRASH)

    z64 = jnp.zeros((RPT, H), jnp.float32)
    z16 = jnp.zeros((RPT, L), jnp.float32)
    ones_h = jnp.ones((K, L), jnp.float32)
    bt = b_tweet.reshape(1, D)
    bi = b_in.reshape(1, D)
    br = b_rgcn.reshape(1, D)
    bo = b_out.reshape(1, D)
    bf = b_fin.reshape(1, 2)
    Wroot2 = W_root.reshape(2, H, D)
    Wrel2 = W_rel.reshape(R, 2, H, D).transpose(1, 0, 2, 3)
    Wout2 = W_out.reshape(2, H, D)

    x1 = _tc_input(tweet, W_tweet, bt, W_in, bi)
    cnt = _sc_cnt(dstc, ones_h, z16).reshape(NC, R, NP, L)

    agg1 = _sc_agg(x1.reshape(NC * N, H), srcb, dstb, z64)
    x2 = _tc_combine(x1, agg1.reshape(NC, R, NP, H), cnt, Wroot2, Wrel2, br)

    agg2 = _sc_agg(x2.reshape(NC * N, H), srcb, dstb, z64)
    x3 = _tc_combine(x2, agg2.reshape(NC, R, NP, H), cnt, Wroot2, Wrel2, br)

    return _tc_final(x3, Wout2, bo, W_fin, bf)


# speedup vs baseline: 5.6206x; 1.0959x over previous
import jax
import jax.numpy as jnp
from jax import lax
from jax.experimental import pallas as pl
from jax.experimental.pallas import tpu as pltpu
from jax.experimental.pallas import tpu_sc as plsc

N = 10000
E = 320000
D = 128
T = 768
R = 2
H = D // 2
NC = 2
NS = 16
L = 16
NP = 10240
K = 128
SP_A = 160
SST = 40
NSTAGE = SP_A // SST
KC = 80
SP_C = 128
RPT = (R * NP) // NS
TRASH = N
BN = 1000
GRID = N // BN


def _leaky(x):
    return jnp.where(x > 0, x, 0.01 * x)


def _sc_agg_body(xcat, srcb, dstb, z64, agg_o,
                 src_v, dst_v, rows0, rows1, sem0, sem1, acc_sp):
    cid = lax.axis_index("c")
    sid = lax.axis_index("s")
    row0 = sid * RPT

    def drain(sem, buf):
        pltpu.make_async_copy(xcat.at[pl.ds(0, K)], buf, sem).wait()

    pltpu.sync_copy(z64, acc_sp.at[pl.ds(row0, RPT)])
    plsc.subcore_barrier()

    for s in range(NSTAGE):
        pltpu.sync_copy(srcb.at[cid, sid, pl.ds(s * SST, SST + 1)], src_v)
        pltpu.sync_copy(dstb.at[sid, pl.ds(s * SST, SST)], dst_v)

        pltpu.async_copy(xcat.at[src_v.at[0]], rows0, sem0)

        def step(i, c):
            j = 2 * i
            pltpu.async_copy(xcat.at[src_v.at[j + 1]], rows1, sem1)
            drain(sem0, rows0)
            pltpu.sync_copy(rows0, acc_sp.at[dst_v.at[j]], add=True)
            pltpu.async_copy(xcat.at[src_v.at[j + 2]], rows0, sem0)
            drain(sem1, rows1)
            pltpu.sync_copy(rows1, acc_sp.at[dst_v.at[j + 1]], add=True)
            return c
        lax.fori_loop(0, SST // 2, step, 0)
        drain(sem0, rows0)

    plsc.subcore_barrier()

    pltpu.sync_copy(acc_sp.at[pl.ds(row0, RPT)],
                    agg_o.at[cid, pl.ds(row0, RPT)])


_sc_agg = pl.kernel(
    _sc_agg_body,
    out_type=jax.ShapeDtypeStruct((NC, R * NP, H), jnp.float32),
    mesh=plsc.VectorSubcoreMesh(core_axis_name="c", subcore_axis_name="s"),
    scratch_types=(
        pltpu.VMEM((SST + 1, K), jnp.int32),
        pltpu.VMEM((SST, K), jnp.int32),
        pltpu.VMEM((K, H), jnp.float32),
        pltpu.VMEM((K, H), jnp.float32),
        pltpu.SemaphoreType.DMA,
        pltpu.SemaphoreType.DMA,
        pltpu.VMEM_SHARED((R * NP, H), jnp.float32),
    ),
    compiler_params=pltpu.CompilerParams(use_tc_tiling_on_sc=False),
)


def _sc_cnt_body(dstc, ones_h, z16, cnt_o, dst_v, ones_v, cnt_sp):
    cid = lax.axis_index("c")
    sid = lax.axis_index("s")
    row0 = sid * RPT

    pltpu.sync_copy(z16, cnt_sp.at[pl.ds(row0, RPT)])
    pltpu.sync_copy(ones_h, ones_v)
    pltpu.sync_copy(dstc.at[cid, sid], dst_v)
    plsc.subcore_barrier()

    def step(j, c):
        pltpu.sync_copy(ones_v, cnt_sp.at[dst_v.at[j]], add=True)
        return c
    lax.fori_loop(0, SP_C, step, 0)

    plsc.subcore_barrier()
    pltpu.sync_copy(cnt_sp.at[pl.ds(row0, RPT)],
                    cnt_o.at[cid, pl.ds(row0, RPT)])


_sc_cnt = pl.kernel(
    _sc_cnt_body,
    out_type=jax.ShapeDtypeStruct((NC, R * NP, L), jnp.float32),
    mesh=plsc.VectorSubcoreMesh(core_axis_name="c", subcore_axis_name="s"),
    scratch_types=(
        pltpu.VMEM((SP_C, KC), jnp.int32),
        pltpu.VMEM((KC, L), jnp.float32),
        pltpu.VMEM_SHARED((R * NP, L), jnp.float32),
    ),
    compiler_params=pltpu.CompilerParams(use_tc_tiling_on_sc=False),
)


def _tc_input(tweet, Wt, bt, Wi, bi):
    def body(tw, wt, bt_, wi, bi_, out):
        x = jnp.dot(tw[...], wt[...], preferred_element_type=jnp.float32)
        x = _leaky(x + bt_[...])
        x = jnp.dot(x, wi[...], preferred_element_type=jnp.float32)
        x = _leaky(x + bi_[...])
        out[0] = x[:, :H]
        out[1] = x[:, H:]

    return pl.pallas_call(
        body,
        grid=(GRID,),
        in_specs=[
            pl.BlockSpec((BN, T), lambda i: (i, 0)),
            pl.BlockSpec((T, D), lambda i: (0, 0)),
            pl.BlockSpec((1, D), lambda i: (0, 0)),
            pl.BlockSpec((D, D), lambda i: (0, 0)),
            pl.BlockSpec((1, D), lambda i: (0, 0)),
        ],
        out_specs=pl.BlockSpec((2, BN, H), lambda i: (0, i, 0)),
        out_shape=jax.ShapeDtypeStruct((2, N, H), jnp.float32),
    )(tweet, Wt, bt, Wi, bi)


def _tc_combine(xs, agg, cnt, Wroot2, Wrel2, b):
    def body(x, a, c, wr, wl, bb, out):
        acc = (jnp.dot(x[0], wr[0], preferred_element_type=jnp.float32)
               + jnp.dot(x[1], wr[1], preferred_element_type=jnp.float32)
               + bb[...])
        for r in range(R):
            s = (jnp.dot(a[0, r], wl[0, r], preferred_element_type=jnp.float32)
                 + jnp.dot(a[1, r], wl[1, r], preferred_element_type=jnp.float32))
            den = c[0, r, :, 0:1] + c[1, r, :, 0:1]
            acc = acc + s / jnp.maximum(den, 1.0)
        out[0] = acc[:, :H]
        out[1] = acc[:, H:]

    return pl.pallas_call(
        body,
        grid=(GRID,),
        in_specs=[
            pl.BlockSpec((2, BN, H), lambda i: (0, i, 0)),
            pl.BlockSpec((2, R, BN, H), lambda i: (0, 0, i, 0)),
            pl.BlockSpec((2, R, BN, L), lambda i: (0, 0, i, 0)),
            pl.BlockSpec((2, H, D), lambda i: (0, 0, 0)),
            pl.BlockSpec((2, R, H, D), lambda i: (0, 0, 0, 0)),
            pl.BlockSpec((1, D), lambda i: (0, 0)),
        ],
        out_specs=pl.BlockSpec((2, BN, H), lambda i: (0, i, 0)),
        out_shape=jax.ShapeDtypeStruct((2, N, H), jnp.float32),
    )(xs, agg, cnt, Wroot2, Wrel2, b)


def _tc_final(xs, Wout2, bo, Wfin, bf):
    def body(x, wo, bo_, wf, bf_, out):
        h = (jnp.dot(x[0], wo[0], preferred_element_type=jnp.float32)
             + jnp.dot(x[1], wo[1], preferred_element_type=jnp.float32)
             + bo_[...])
        h = _leaky(h)
        out[...] = jnp.dot(h, wf[...], preferred_element_type=jnp.float32) + bf_[...]

    return pl.pallas_call(
        body,
        grid=(GRID,),
        in_specs=[
            pl.BlockSpec((2, BN, H), lambda i: (0, i, 0)),
            pl.BlockSpec((2, H, D), lambda i: (0, 0, 0)),
            pl.BlockSpec((1, D), lambda i: (0, 0)),
            pl.BlockSpec((D, 2), lambda i: (0, 0)),
            pl.BlockSpec((1, 2), lambda i: (0, 0)),
        ],
        out_specs=pl.BlockSpec((BN, 2), lambda i: (i, 0)),
        out_shape=jax.ShapeDtypeStruct((N, 2), jnp.float32),
    )(xs, Wout2, bo, Wfin, bf)


def kernel(tweet, edge_index, edge_type, W_tweet, b_tweet, W_in, b_in,
           W_rel, W_root, b_rgcn, W_out, b_out, W_fin, b_fin):
    src = edge_index[0].astype(jnp.int32)
    dstp = edge_index[1].astype(jnp.int32) + edge_type.astype(jnp.int32) * NP

    pe = NS * SP_A * K - E
    srcp = jnp.pad(src, (0, pe)).reshape(NS, SP_A, K)
    srcp = jnp.pad(srcp, ((0, 0), (0, 8), (0, 0)))
    srcb = srcp[None, :, :, :] + (jnp.arange(NC, dtype=jnp.int32) * N
                                  ).reshape(NC, 1, 1, 1)
    dstb = jnp.pad(dstp, (0, pe), constant_values=TRASH).reshape(NS, SP_A, K)

    dstc = dstp.reshape(NC, NS, SP_C - 3, KC)
    dstc = jnp.pad(dstc, ((0, 0), (0, 0), (0, 3), (0, 0)),
                   constant_values=TRASH)

    z64 = jnp.zeros((RPT, H), jnp.float32)
    z16 = jnp.zeros((RPT, L), jnp.float32)
    ones_h = jnp.ones((KC, L), jnp.float32)
    bt = b_tweet.reshape(1, D)
    bi = b_in.reshape(1, D)
    br = b_rgcn.reshape(1, D)
    bo = b_out.reshape(1, D)
    bf = b_fin.reshape(1, 2)
    Wroot2 = W_root.reshape(2, H, D)
    Wrel2 = W_rel.reshape(R, 2, H, D).transpose(1, 0, 2, 3)
    Wout2 = W_out.reshape(2, H, D)

    x1 = _tc_input(tweet, W_tweet, bt, W_in, bi)
    cnt = _sc_cnt(dstc, ones_h, z16).reshape(NC, R, NP, L)

    agg1 = _sc_agg(x1.reshape(NC * N, H), srcb, dstb, z64)
    x2 = _tc_combine(x1, agg1.reshape(NC, R, NP, H), cnt, Wroot2, Wrel2, br)

    agg2 = _sc_agg(x2.reshape(NC * N, H), srcb, dstb, z64)
    x3 = _tc_combine(x2, agg2.reshape(NC, R, NP, H), cnt, Wroot2, Wrel2, br)

    return _tc_final(x3, Wout2, bo, W_fin, bf)


# speedup vs baseline: 6.2696x; 1.1155x over previous
import jax
import jax.numpy as jnp
from jax import lax
from jax.experimental import pallas as pl
from jax.experimental.pallas import tpu as pltpu
from jax.experimental.pallas import tpu_sc as plsc

N = 10000
E = 320000
D = 128
T = 768
R = 2
H = D // 2
NC = 2
NS = 16
L = 16
NP = 10240
K = 128
SP_A = 160
SST = 32
NSTAGE = SP_A // SST
NB = 4
KC = 80
SP_C = 128
RPT = (R * NP) // NS
TRASH = N
BN = 1000
GRID = N // BN


def _leaky(x):
    return jnp.where(x > 0, x, 0.01 * x)


def _sc_agg_body(xcat, srcb, dstb, z64, agg_o,
                 src_v, dst_v, r0, r1, r2, r3, s0, s1, s2, s3, acc_sp):
    cid = lax.axis_index("c")
    sid = lax.axis_index("s")
    row0 = sid * RPT
    rows = (r0, r1, r2, r3)
    sems = (s0, s1, s2, s3)

    def drain(sem, buf):
        pltpu.make_async_copy(xcat.at[pl.ds(0, K)], buf, sem).wait()

    pltpu.sync_copy(z64, acc_sp.at[pl.ds(row0, RPT)])
    plsc.subcore_barrier()

    for s in range(NSTAGE):
        pltpu.sync_copy(srcb.at[cid, sid, pl.ds(s * SST, SST)], src_v)
        pltpu.sync_copy(dstb.at[sid, pl.ds(s * SST, SST)], dst_v)

        for b in range(NB):
            pltpu.async_copy(xcat.at[src_v.at[b]], rows[b], sems[b])

        def step(i, c):
            j = NB * i
            for b in range(NB):
                drain(sems[b], rows[b])
                pltpu.sync_copy(rows[b], acc_sp.at[dst_v.at[j + b]], add=True)
                pltpu.async_copy(xcat.at[src_v.at[j + b + NB]], rows[b],
                                 sems[b])
            return c
        lax.fori_loop(0, SST // NB - 1, step, 0)

        for b in range(NB):
            drain(sems[b], rows[b])
            pltpu.sync_copy(rows[b], acc_sp.at[dst_v.at[SST - NB + b]],
                            add=True)

    plsc.subcore_barrier()

    pltpu.sync_copy(acc_sp.at[pl.ds(row0, RPT)],
                    agg_o.at[cid, pl.ds(row0, RPT)])


_sc_agg = pl.kernel(
    _sc_agg_body,
    out_type=jax.ShapeDtypeStruct((NC, R * NP, H), jnp.float32),
    mesh=plsc.VectorSubcoreMesh(core_axis_name="c", subcore_axis_name="s"),
    scratch_types=(
        pltpu.VMEM((SST, K), jnp.int32),
        pltpu.VMEM((SST, K), jnp.int32),
        pltpu.VMEM((K, H), jnp.float32),
        pltpu.VMEM((K, H), jnp.float32),
        pltpu.VMEM((K, H), jnp.float32),
        pltpu.VMEM((K, H), jnp.float32),
        pltpu.SemaphoreType.DMA,
        pltpu.SemaphoreType.DMA,
        pltpu.SemaphoreType.DMA,
        pltpu.SemaphoreType.DMA,
        pltpu.VMEM_SHARED((R * NP, H), jnp.float32),
    ),
    compiler_params=pltpu.CompilerParams(use_tc_tiling_on_sc=False),
)


def _sc_cnt_body(dstc, ones_h, z16, cnt_o, dst_v, ones_v, cnt_sp):
    cid = lax.axis_index("c")
    sid = lax.axis_index("s")
    row0 = sid * RPT

    pltpu.sync_copy(z16, cnt_sp.at[pl.ds(row0, RPT)])
    pltpu.sync_copy(ones_h, ones_v)
    pltpu.sync_copy(dstc.at[cid, sid], dst_v)
    plsc.subcore_barrier()

    def step(j, c):
        pltpu.sync_copy(ones_v, cnt_sp.at[dst_v.at[j]], add=True)
        return c
    lax.fori_loop(0, SP_C, step, 0)

    plsc.subcore_barrier()
    pltpu.sync_copy(cnt_sp.at[pl.ds(row0, RPT)],
                    cnt_o.at[cid, pl.ds(row0, RPT)])


_sc_cnt = pl.kernel(
    _sc_cnt_body,
    out_type=jax.ShapeDtypeStruct((NC, R * NP, L), jnp.float32),
    mesh=plsc.VectorSubcoreMesh(core_axis_name="c", subcore_axis_name="s"),
    scratch_types=(
        pltpu.VMEM((SP_C, KC), jnp.int32),
        pltpu.VMEM((KC, L), jnp.float32),
        pltpu.VMEM_SHARED((R * NP, L), jnp.float32),
    ),
    compiler_params=pltpu.CompilerParams(use_tc_tiling_on_sc=False),
)


def _tc_input(tweet, Wt, bt, Wi, bi):
    def body(tw, wt, bt_, wi, bi_, out):
        x = jnp.dot(tw[...], wt[...], preferred_element_type=jnp.float32)
        x = _leaky(x + bt_[...])
        x = jnp.dot(x, wi[...], preferred_element_type=jnp.float32)
        x = _leaky(x + bi_[...])
        out[0] = x[:, :H]
        out[1] = x[:, H:]

    return pl.pallas_call(
        body,
        grid=(GRID,),
        in_specs=[
            pl.BlockSpec((BN, T), lambda i: (i, 0)),
            pl.BlockSpec((T, D), lambda i: (0, 0)),
            pl.BlockSpec((1, D), lambda i: (0, 0)),
            pl.BlockSpec((D, D), lambda i: (0, 0)),
            pl.BlockSpec((1, D), lambda i: (0, 0)),
        ],
        out_specs=pl.BlockSpec((2, BN, H), lambda i: (0, i, 0)),
        out_shape=jax.ShapeDtypeStruct((2, N, H), jnp.float32),
    )(tweet, Wt, bt, Wi, bi)


def _tc_combine(xs, agg, cnt, Wroot2, Wrel2, b):
    def body(x, a, c, wr, wl, bb, out):
        acc = (jnp.dot(x[0], wr[0], preferred_element_type=jnp.float32)
               + jnp.dot(x[1], wr[1], preferred_element_type=jnp.float32)
               + bb[...])
        for r in range(R):
            s = (jnp.dot(a[0, r], wl[0, r], preferred_element_type=jnp.float32)
                 + jnp.dot(a[1, r], wl[1, r], preferred_element_type=jnp.float32))
            den = c[0, r, :, 0:1] + c[1, r, :, 0:1]
            acc = acc + s / jnp.maximum(den, 1.0)
        out[0] = acc[:, :H]
        out[1] = acc[:, H:]

    return pl.pallas_call(
        body,
        grid=(GRID,),
        in_specs=[
            pl.BlockSpec((2, BN, H), lambda i: (0, i, 0)),
            pl.BlockSpec((2, R, BN, H), lambda i: (0, 0, i, 0)),
            pl.BlockSpec((2, R, BN, L), lambda i: (0, 0, i, 0)),
            pl.BlockSpec((2, H, D), lambda i: (0, 0, 0)),
            pl.BlockSpec((2, R, H, D), lambda i: (0, 0, 0, 0)),
            pl.BlockSpec((1, D), lambda i: (0, 0)),
        ],
        out_specs=pl.BlockSpec((2, BN, H), lambda i: (0, i, 0)),
        out_shape=jax.ShapeDtypeStruct((2, N, H), jnp.float32),
    )(xs, agg, cnt, Wroot2, Wrel2, b)


def _tc_final(xs, Wout2, bo, Wfin, bf):
    def body(x, wo, bo_, wf, bf_, out):
        h = (jnp.dot(x[0], wo[0], preferred_element_type=jnp.float32)
             + jnp.dot(x[1], wo[1], preferred_element_type=jnp.float32)
             + bo_[...])
        h = _leaky(h)
        out[...] = jnp.dot(h, wf[...], preferred_element_type=jnp.float32) + bf_[...]

    return pl.pallas_call(
        body,
        grid=(GRID,),
        in_specs=[
            pl.BlockSpec((2, BN, H), lambda i: (0, i, 0)),
            pl.BlockSpec((2, H, D), lambda i: (0, 0, 0)),
            pl.BlockSpec((1, D), lambda i: (0, 0)),
            pl.BlockSpec((D, 2), lambda i: (0, 0)),
            pl.BlockSpec((1, 2), lambda i: (0, 0)),
        ],
        out_specs=pl.BlockSpec((BN, 2), lambda i: (i, 0)),
        out_shape=jax.ShapeDtypeStruct((N, 2), jnp.float32),
    )(xs, Wout2, bo, Wfin, bf)


def kernel(tweet, edge_index, edge_type, W_tweet, b_tweet, W_in, b_in,
           W_rel, W_root, b_rgcn, W_out, b_out, W_fin, b_fin):
    src = edge_index[0].astype(jnp.int32)
    dstp = edge_index[1].astype(jnp.int32) + edge_type.astype(jnp.int32) * NP

    pe = NS * SP_A * K - E
    srcp = jnp.pad(src, (0, pe)).reshape(NS, SP_A, K)
    srcb = srcp[None, :, :, :] + (jnp.arange(NC, dtype=jnp.int32) * N
                                  ).reshape(NC, 1, 1, 1)
    dstb = jnp.pad(dstp, (0, pe), constant_values=TRASH).reshape(NS, SP_A, K)

    dstc = dstp.reshape(NC, NS, SP_C - 3, KC)
    dstc = jnp.pad(dstc, ((0, 0), (0, 0), (0, 3), (0, 0)),
                   constant_values=TRASH)

    z64 = jnp.zeros((RPT, H), jnp.float32)
    z16 = jnp.zeros((RPT, L), jnp.float32)
    ones_h = jnp.ones((KC, L), jnp.float32)
    bt = b_tweet.reshape(1, D)
    bi = b_in.reshape(1, D)
    br = b_rgcn.reshape(1, D)
    bo = b_out.reshape(1, D)
    bf = b_fin.reshape(1, 2)
    Wroot2 = W_root.reshape(2, H, D)
    Wrel2 = W_rel.reshape(R, 2, H, D).transpose(1, 0, 2, 3)
    Wout2 = W_out.reshape(2, H, D)

    x1 = _tc_input(tweet, W_tweet, bt, W_in, bi)
    cnt = _sc_cnt(dstc, ones_h, z16).reshape(NC, R, NP, L)

    agg1 = _sc_agg(x1.reshape(NC * N, H), srcb, dstb, z64)
    x2 = _tc_combine(x1, agg1.reshape(NC, R, NP, H), cnt, Wroot2, Wrel2, br)

    agg2 = _sc_agg(x2.reshape(NC * N, H), srcb, dstb, z64)
    x3 = _tc_combine(x2, agg2.reshape(NC, R, NP, H), cnt, Wroot2, Wrel2, br)

    return _tc_final(x3, Wout2, bo, W_fin, bf)


# speedup vs baseline: 6.2729x; 1.0005x over previous
import jax
import jax.numpy as jnp
from jax import lax
from jax.experimental import pallas as pl
from jax.experimental.pallas import tpu as pltpu
from jax.experimental.pallas import tpu_sc as plsc

N = 10000
E = 320000
D = 128
T = 768
R = 2
H = D // 2
NC = 2
NS = 16
L = 16
NP = 10240
K = 128
SP_A = 160
SST = 32
NSTAGE = SP_A // SST
NB = 4
KC = 80
SP_C = 128
RPT = (R * NP) // NS
TRASH = N
BN = 1000
GRID = N // BN


def _leaky(x):
    return jnp.where(x > 0, x, 0.01 * x)


def _sc_agg_body(xcat, srcb, dstb, z64, agg_o,
                 src_v, dst_v, r0, r1, r2, r3,
                 g0, g1, g2, g3, acc_sp):
    cid = lax.axis_index("c")
    sid = lax.axis_index("s")
    row0 = sid * RPT
    rows = (r0, r1, r2, r3)
    semg = (g0, g1, g2, g3)

    def wait_g(b):
        pltpu.make_async_copy(xcat.at[pl.ds(0, K)], rows[b], semg[b]).wait()

    def start_g(idx, b):
        pltpu.async_copy(xcat.at[src_v.at[idx]], rows[b], semg[b])

    def scat(idx, b):
        pltpu.sync_copy(rows[b], acc_sp.at[dst_v.at[idx]], add=True)

    pltpu.sync_copy(z64, acc_sp.at[pl.ds(row0, RPT)])
    plsc.subcore_barrier()

    for s in range(NSTAGE):
        pltpu.sync_copy(srcb.at[cid, sid, pl.ds(s * SST, SST)], src_v)
        pltpu.sync_copy(dstb.at[sid, pl.ds(s * SST, SST)], dst_v)

        for b in range(NB):
            start_g(b, b)

        def step(i, c):
            for b in range(NB):
                cs = NB * i + b
                wait_g(b)
                scat(cs, b)
                start_g(cs + NB, b)
            return c
        lax.fori_loop(0, SST // NB - 1, step, 0)

        for b in range(NB):
            wait_g(b)
            scat(SST - NB + b, b)

    plsc.subcore_barrier()

    pltpu.sync_copy(acc_sp.at[pl.ds(row0, RPT)],
                    agg_o.at[cid, pl.ds(row0, RPT)])


_sc_agg = pl.kernel(
    _sc_agg_body,
    out_type=jax.ShapeDtypeStruct((NC, R * NP, H), jnp.float32),
    mesh=plsc.VectorSubcoreMesh(core_axis_name="c", subcore_axis_name="s"),
    scratch_types=(
        pltpu.VMEM((SST, K), jnp.int32),
        pltpu.VMEM((SST, K), jnp.int32),
        pltpu.VMEM((K, H), jnp.float32),
        pltpu.VMEM((K, H), jnp.float32),
        pltpu.VMEM((K, H), jnp.float32),
        pltpu.VMEM((K, H), jnp.float32),
        pltpu.SemaphoreType.DMA,
        pltpu.SemaphoreType.DMA,
        pltpu.SemaphoreType.DMA,
        pltpu.SemaphoreType.DMA,
        pltpu.VMEM_SHARED((R * NP, H), jnp.float32),
    ),
    compiler_params=pltpu.CompilerParams(use_tc_tiling_on_sc=False),
)


def _sc_cnt_body(dstc, ones_h, z16, cnt_o, dst_v, ones_v, cnt_sp):
    cid = lax.axis_index("c")
    sid = lax.axis_index("s")
    row0 = sid * RPT

    pltpu.sync_copy(z16, cnt_sp.at[pl.ds(row0, RPT)])
    pltpu.sync_copy(ones_h, ones_v)
    pltpu.sync_copy(dstc.at[cid, sid], dst_v)
    plsc.subcore_barrier()

    def step(j, c):
        pltpu.sync_copy(ones_v, cnt_sp.at[dst_v.at[j]], add=True)
        return c
    lax.fori_loop(0, SP_C, step, 0)

    plsc.subcore_barrier()
    pltpu.sync_copy(cnt_sp.at[pl.ds(row0, RPT)],
                    cnt_o.at[cid, pl.ds(row0, RPT)])


_sc_cnt = pl.kernel(
    _sc_cnt_body,
    out_type=jax.ShapeDtypeStruct((NC, R * NP, L), jnp.float32),
    mesh=plsc.VectorSubcoreMesh(core_axis_name="c", subcore_axis_name="s"),
    scratch_types=(
        pltpu.VMEM((SP_C, KC), jnp.int32),
        pltpu.VMEM((KC, L), jnp.float32),
        pltpu.VMEM_SHARED((R * NP, L), jnp.float32),
    ),
    compiler_params=pltpu.CompilerParams(use_tc_tiling_on_sc=False),
)


def _tc_input(tweet, Wt, bt, Wi, bi):
    def body(tw, wt, bt_, wi, bi_, out):
        x = jnp.dot(tw[...], wt[...], preferred_element_type=jnp.float32)
        x = _leaky(x + bt_[...])
        x = jnp.dot(x, wi[...], preferred_element_type=jnp.float32)
        x = _leaky(x + bi_[...])
        out[0] = x[:, :H]
        out[1] = x[:, H:]

    return pl.pallas_call(
        body,
        grid=(GRID,),
        in_specs=[
            pl.BlockSpec((BN, T), lambda i: (i, 0)),
            pl.BlockSpec((T, D), lambda i: (0, 0)),
            pl.BlockSpec((1, D), lambda i: (0, 0)),
            pl.BlockSpec((D, D), lambda i: (0, 0)),
            pl.BlockSpec((1, D), lambda i: (0, 0)),
        ],
        out_specs=pl.BlockSpec((2, BN, H), lambda i: (0, i, 0)),
        out_shape=jax.ShapeDtypeStruct((2, N, H), jnp.float32),
    )(tweet, Wt, bt, Wi, bi)


def _tc_combine(xs, agg, cnt, Wroot2, Wrel2, b):
    def body(x, a, c, wr, wl, bb, out):
        acc = (jnp.dot(x[0], wr[0], preferred_element_type=jnp.float32)
               + jnp.dot(x[1], wr[1], preferred_element_type=jnp.float32)
               + bb[...])
        for r in range(R):
            s = (jnp.dot(a[0, r], wl[0, r], preferred_element_type=jnp.float32)
                 + jnp.dot(a[1, r], wl[1, r], preferred_element_type=jnp.float32))
            den = c[0, r, :, 0:1] + c[1, r, :, 0:1]
            acc = acc + s / jnp.maximum(den, 1.0)
        out[0] = acc[:, :H]
        out[1] = acc[:, H:]

    return pl.pallas_call(
        body,
        grid=(GRID,),
        in_specs=[
            pl.BlockSpec((2, BN, H), lambda i: (0, i, 0)),
            pl.BlockSpec((2, R, BN, H), lambda i: (0, 0, i, 0)),
            pl.BlockSpec((2, R, BN, L), lambda i: (0, 0, i, 0)),
            pl.BlockSpec((2, H, D), lambda i: (0, 0, 0)),
            pl.BlockSpec((2, R, H, D), lambda i: (0, 0, 0, 0)),
            pl.BlockSpec((1, D), lambda i: (0, 0)),
        ],
        out_specs=pl.BlockSpec((2, BN, H), lambda i: (0, i, 0)),
        out_shape=jax.ShapeDtypeStruct((2, N, H), jnp.float32),
    )(xs, agg, cnt, Wroot2, Wrel2, b)


def _tc_final(xs, Wout2, bo, Wfin, bf):
    def body(x, wo, bo_, wf, bf_, out):
        h = (jnp.dot(x[0], wo[0], preferred_element_type=jnp.float32)
             + jnp.dot(x[1], wo[1], preferred_element_type=jnp.float32)
             + bo_[...])
        h = _leaky(h)
        out[...] = jnp.dot(h, wf[...], preferred_element_type=jnp.float32) + bf_[...]

    return pl.pallas_call(
        body,
        grid=(GRID,),
        in_specs=[
            pl.BlockSpec((2, BN, H), lambda i: (0, i, 0)),
            pl.BlockSpec((2, H, D), lambda i: (0, 0, 0)),
            pl.BlockSpec((1, D), lambda i: (0, 0)),
            pl.BlockSpec((D, 2), lambda i: (0, 0)),
            pl.BlockSpec((1, 2), lambda i: (0, 0)),
        ],
        out_specs=pl.BlockSpec((BN, 2), lambda i: (i, 0)),
        out_shape=jax.ShapeDtypeStruct((N, 2), jnp.float32),
    )(xs, Wout2, bo, Wfin, bf)


def kernel(tweet, edge_index, edge_type, W_tweet, b_tweet, W_in, b_in,
           W_rel, W_root, b_rgcn, W_out, b_out, W_fin, b_fin):
    src = edge_index[0].astype(jnp.int32)
    dstp = edge_index[1].astype(jnp.int32) + edge_type.astype(jnp.int32) * NP

    pe = NS * SP_A * K - E
    srcp = jnp.pad(src, (0, pe)).reshape(NS, SP_A, K)
    srcb = srcp[None, :, :, :] + (jnp.arange(NC, dtype=jnp.int32) * N
                                  ).reshape(NC, 1, 1, 1)
    dstb = jnp.pad(dstp, (0, pe), constant_values=TRASH).reshape(NS, SP_A, K)

    dstc = dstp.reshape(NC, NS, SP_C - 3, KC)
    dstc = jnp.pad(dstc, ((0, 0), (0, 0), (0, 3), (0, 0)),
                   constant_values=TRASH)

    z64 = jnp.zeros((RPT, H), jnp.float32)
    z16 = jnp.zeros((RPT, L), jnp.float32)
    ones_h = jnp.ones((KC, L), jnp.float32)
    bt = b_tweet.reshape(1, D)
    bi = b_in.reshape(1, D)
    br = b_rgcn.reshape(1, D)
    bo = b_out.reshape(1, D)
    bf = b_fin.reshape(1, 2)
    Wroot2 = W_root.reshape(2, H, D)
    Wrel2 = W_rel.reshape(R, 2, H, D).transpose(1, 0, 2, 3)
    Wout2 = W_out.reshape(2, H, D)

    x1 = _tc_input(tweet, W_tweet, bt, W_in, bi)
    cnt = _sc_cnt(dstc, ones_h, z16).reshape(NC, R, NP, L)

    agg1 = _sc_agg(x1.reshape(NC * N, H), srcb, dstb, z64)
    x2 = _tc_combine(x1, agg1.reshape(NC, R, NP, H), cnt, Wroot2, Wrel2, br)

    agg2 = _sc_agg(x2.reshape(NC * N, H), srcb, dstb, z64)
    x3 = _tc_combine(x2, agg2.reshape(NC, R, NP, H), cnt, Wroot2, Wrel2, br)

    return _tc_final(x3, Wout2, bo, W_fin, bf)


# speedup vs baseline: 6.2992x; 1.0042x over previous
import jax
import jax.numpy as jnp
from jax import lax
from jax.experimental import pallas as pl
from jax.experimental.pallas import tpu as pltpu
from jax.experimental.pallas import tpu_sc as plsc

N = 10000
E = 320000
D = 128
T = 768
R = 2
H = D // 2
NC = 2
NS = 16
L = 16
NP = 10240
K = 128
SP_A = 160
SST = 32
NSTAGE = SP_A // SST
NB = 4
KC = 80
SP_C = 128
RPT = (R * NP) // NS
TRASH = N
BN = 1000
GRID = N // BN


def _leaky(x):
    return jnp.where(x > 0, x, 0.01 * x)


def _sc_agg_body(xcat, srcb, dstb, z64, agg_o,
                 src_v, dst_v, r0, r1, r2, r3,
                 g0, g1, g2, g3, t0, acc_sp):
    cid = lax.axis_index("c")
    sid = lax.axis_index("s")
    row0 = sid * RPT
    rows = (r0, r1, r2, r3)
    semg = (g0, g1, g2, g3)

    def wait_g(b):
        pltpu.make_async_copy(xcat.at[pl.ds(0, K)], rows[b], semg[b]).wait()

    def wait_s():
        pltpu.make_async_copy(rows[0], acc_sp.at[pl.ds(0, K)], t0).wait()

    def start_g(idx, b):
        pltpu.async_copy(xcat.at[src_v.at[idx]], rows[b], semg[b])

    def start_s(idx, b):
        pltpu.async_copy(rows[b], acc_sp.at[dst_v.at[idx]], t0, add=True)

    pltpu.sync_copy(z64, acc_sp.at[pl.ds(row0, RPT)])
    plsc.subcore_barrier()

    for s in range(NSTAGE):
        pltpu.sync_copy(srcb.at[cid, sid, pl.ds(s * SST, SST)], src_v)
        pltpu.sync_copy(dstb.at[sid, pl.ds(s * SST, SST)], dst_v)

        start_g(0, 0)
        start_g(1, 1)
        start_g(2, 2)
        wait_g(0)
        start_s(0, 0)
        start_g(3, 3)

        def step(i, c):
            for j in range(NB):
                cs = NB * i + 1 + j
                b = (1 + j) % NB
                wait_g(b)
                wait_s()
                start_s(cs, b)
                start_g(cs + 3, (b + 3) % NB)
            return c
        lax.fori_loop(0, (SST - NB) // NB, step, 0)

        for cs in range(SST - 3, SST):
            wait_g(cs % NB)
            wait_s()
            start_s(cs, cs % NB)
        wait_s()

    plsc.subcore_barrier()

    pltpu.sync_copy(acc_sp.at[pl.ds(row0, RPT)],
                    agg_o.at[cid, pl.ds(row0, RPT)])


_sc_agg = pl.kernel(
    _sc_agg_body,
    out_type=jax.ShapeDtypeStruct((NC, R * NP, H), jnp.float32),
    mesh=plsc.VectorSubcoreMesh(core_axis_name="c", subcore_axis_name="s"),
    scratch_types=(
        pltpu.VMEM((SST, K), jnp.int32),
        pltpu.VMEM((SST, K), jnp.int32),
        pltpu.VMEM((K, H), jnp.float32),
        pltpu.VMEM((K, H), jnp.float32),
        pltpu.VMEM((K, H), jnp.float32),
        pltpu.VMEM((K, H), jnp.float32),
        pltpu.SemaphoreType.DMA,
        pltpu.SemaphoreType.DMA,
        pltpu.SemaphoreType.DMA,
        pltpu.SemaphoreType.DMA,
        pltpu.SemaphoreType.DMA,
        pltpu.VMEM_SHARED((R * NP, H), jnp.float32),
    ),
    compiler_params=pltpu.CompilerParams(use_tc_tiling_on_sc=False),
)


def _sc_cnt_body(dstc, ones_h, z16, cnt_o, dst_v, ones_v, cnt_sp):
    cid = lax.axis_index("c")
    sid = lax.axis_index("s")
    row0 = sid * RPT

    pltpu.sync_copy(z16, cnt_sp.at[pl.ds(row0, RPT)])
    pltpu.sync_copy(ones_h, ones_v)
    pltpu.sync_copy(dstc.at[cid, sid], dst_v)
    plsc.subcore_barrier()

    def step(j, c):
        pltpu.sync_copy(ones_v, cnt_sp.at[dst_v.at[j]], add=True)
        return c
    lax.fori_loop(0, SP_C, step, 0)

    plsc.subcore_barrier()
    pltpu.sync_copy(cnt_sp.at[pl.ds(row0, RPT)],
                    cnt_o.at[cid, pl.ds(row0, RPT)])


_sc_cnt = pl.kernel(
    _sc_cnt_body,
    out_type=jax.ShapeDtypeStruct((NC, R * NP, L), jnp.float32),
    mesh=plsc.VectorSubcoreMesh(core_axis_name="c", subcore_axis_name="s"),
    scratch_types=(
        pltpu.VMEM((SP_C, KC), jnp.int32),
        pltpu.VMEM((KC, L), jnp.float32),
        pltpu.VMEM_SHARED((R * NP, L), jnp.float32),
    ),
    compiler_params=pltpu.CompilerParams(use_tc_tiling_on_sc=False),
)


def _tc_input(tweet, Wt, bt, Wi, bi):
    def body(tw, wt, bt_, wi, bi_, out):
        x = jnp.dot(tw[...], wt[...], preferred_element_type=jnp.float32)
        x = _leaky(x + bt_[...])
        x = jnp.dot(x, wi[...], preferred_element_type=jnp.float32)
        x = _leaky(x + bi_[...])
        out[0] = x[:, :H]
        out[1] = x[:, H:]

    return pl.pallas_call(
        body,
        grid=(GRID,),
        in_specs=[
            pl.BlockSpec((BN, T), lambda i: (i, 0)),
            pl.BlockSpec((T, D), lambda i: (0, 0)),
            pl.BlockSpec((1, D), lambda i: (0, 0)),
            pl.BlockSpec((D, D), lambda i: (0, 0)),
            pl.BlockSpec((1, D), lambda i: (0, 0)),
        ],
        out_specs=pl.BlockSpec((2, BN, H), lambda i: (0, i, 0)),
        out_shape=jax.ShapeDtypeStruct((2, N, H), jnp.float32),
    )(tweet, Wt, bt, Wi, bi)


def _tc_combine(xs, agg, cnt, Wroot2, Wrel2, b):
    def body(x, a, c, wr, wl, bb, out):
        acc = (jnp.dot(x[0], wr[0], preferred_element_type=jnp.float32)
               + jnp.dot(x[1], wr[1], preferred_element_type=jnp.float32)
               + bb[...])
        for r in range(R):
            s = (jnp.dot(a[0, r], wl[0, r], preferred_element_type=jnp.float32)
                 + jnp.dot(a[1, r], wl[1, r], preferred_element_type=jnp.float32))
            den = c[0, r, :, 0:1] + c[1, r, :, 0:1]
            acc = acc + s / jnp.maximum(den, 1.0)
        out[0] = acc[:, :H]
        out[1] = acc[:, H:]

    return pl.pallas_call(
        body,
        grid=(GRID,),
        in_specs=[
            pl.BlockSpec((2, BN, H), lambda i: (0, i, 0)),
            pl.BlockSpec((2, R, BN, H), lambda i: (0, 0, i, 0)),
            pl.BlockSpec((2, R, BN, L), lambda i: (0, 0, i, 0)),
            pl.BlockSpec((2, H, D), lambda i: (0, 0, 0)),
            pl.BlockSpec((2, R, H, D), lambda i: (0, 0, 0, 0)),
            pl.BlockSpec((1, D), lambda i: (0, 0)),
        ],
        out_specs=pl.BlockSpec((2, BN, H), lambda i: (0, i, 0)),
        out_shape=jax.ShapeDtypeStruct((2, N, H), jnp.float32),
    )(xs, agg, cnt, Wroot2, Wrel2, b)


def _tc_final(xs, Wout2, bo, Wfin, bf):
    def body(x, wo, bo_, wf, bf_, out):
        h = (jnp.dot(x[0], wo[0], preferred_element_type=jnp.float32)
             + jnp.dot(x[1], wo[1], preferred_element_type=jnp.float32)
             + bo_[...])
        h = _leaky(h)
        out[...] = jnp.dot(h, wf[...], preferred_element_type=jnp.float32) + bf_[...]

    return pl.pallas_call(
        body,
        grid=(GRID,),
        in_specs=[
            pl.BlockSpec((2, BN, H), lambda i: (0, i, 0)),
            pl.BlockSpec((2, H, D), lambda i: (0, 0, 0)),
            pl.BlockSpec((1, D), lambda i: (0, 0)),
            pl.BlockSpec((D, 2), lambda i: (0, 0)),
            pl.BlockSpec((1, 2), lambda i: (0, 0)),
        ],
        out_specs=pl.BlockSpec((BN, 2), lambda i: (i, 0)),
        out_shape=jax.ShapeDtypeStruct((N, 2), jnp.float32),
    )(xs, Wout2, bo, Wfin, bf)


def kernel(tweet, edge_index, edge_type, W_tweet, b_tweet, W_in, b_in,
           W_rel, W_root, b_rgcn, W_out, b_out, W_fin, b_fin):
    src = edge_index[0].astype(jnp.int32)
    dstp = edge_index[1].astype(jnp.int32) + edge_type.astype(jnp.int32) * NP

    pe = NS * SP_A * K - E
    srcp = jnp.pad(src, (0, pe)).reshape(NS, SP_A, K)
    srcb = srcp[None, :, :, :] + (jnp.arange(NC, dtype=jnp.int32) * N
                                  ).reshape(NC, 1, 1, 1)
    dstb = jnp.pad(dstp, (0, pe), constant_values=TRASH).reshape(NS, SP_A, K)

    dstc = dstp.reshape(NC, NS, SP_C - 3, KC)
    dstc = jnp.pad(dstc, ((0, 0), (0, 0), (0, 3), (0, 0)),
                   constant_values=TRASH)

    z64 = jnp.zeros((RPT, H), jnp.float32)
    z16 = jnp.zeros((RPT, L), jnp.float32)
    ones_h = jnp.ones((KC, L), jnp.float32)
    bt = b_tweet.reshape(1, D)
    bi = b_in.reshape(1, D)
    br = b_rgcn.reshape(1, D)
    bo = b_out.reshape(1, D)
    bf = b_fin.reshape(1, 2)
    Wroot2 = W_root.reshape(2, H, D)
    Wrel2 = W_rel.reshape(R, 2, H, D).transpose(1, 0, 2, 3)
    Wout2 = W_out.reshape(2, H, D)

    x1 = _tc_input(tweet, W_tweet, bt, W_in, bi)
    cnt = _sc_cnt(dstc, ones_h, z16).reshape(NC, R, NP, L)

    agg1 = _sc_agg(x1.reshape(NC * N, H), srcb, dstb, z64)
    x2 = _tc_combine(x1, agg1.reshape(NC, R, NP, H), cnt, Wroot2, Wrel2, br)

    agg2 = _sc_agg(x2.reshape(NC * N, H), srcb, dstb, z64)
    x3 = _tc_combine(x2, agg2.reshape(NC, R, NP, H), cnt, Wroot2, Wrel2, br)

    return _tc_final(x3, Wout2, bo, W_fin, bf)
